# speedup vs baseline: 1.1154x; 1.1154x over previous
import functools

import jax
import jax.numpy as jnp
from jax.experimental import pallas as pl
from jax.experimental.pallas import tpu as pltpu

GAMMA = 0.1
THRESH = 0.05


def _bi_block_kernel(xyz1_ref, xyz2_ref, points1_ref, points2_ref,
                     wq_ref, wk_ref, wv_ref, wp_ref, out_ref,
                     k2_scr, v_scr):
    nt = pl.program_id(1)
    D = wq_ref.shape[0]

    @pl.when(nt == 0)
    def _():
        p2 = points2_ref[0]
        k2_scr[...] = jnp.dot(p2, wk_ref[...], preferred_element_type=jnp.float32)
        v_scr[...] = jnp.dot(p2, wv_ref[...], preferred_element_type=jnp.float32)

    p1 = points1_ref[0]
    scale = D ** -0.5
    q2 = jnp.dot(p1, wq_ref[...], preferred_element_type=jnp.float32) * scale
    s = jax.lax.dot_general(q2, k2_scr[...], (((1,), (1,)), ((), ())),
                            preferred_element_type=jnp.float32)

    x1 = xyz1_ref[0]
    x2 = xyz2_ref[0]
    dist = ((x1[:, 0:1] - x2[:, 0][None, :]) ** 2
            + (x1[:, 1:2] - x2[:, 1][None, :]) ** 2
            + (x1[:, 2:3] - x2[:, 2][None, :]) ** 2)
    att1 = jnp.exp(-GAMMA * dist)
    att1 = jnp.where(att1 <= THRESH, 0.0, att1)

    m = jnp.max(s, axis=-1, keepdims=True)
    p = jnp.exp(s - m)
    denom = jnp.sum(p, axis=-1, keepdims=True)
    w = att1 * p
    kf = jnp.dot(w, v_scr[...], preferred_element_type=jnp.float32) / denom

    out_ref[0] = (jnp.dot(p1, wp_ref[:D, :], preferred_element_type=jnp.float32)
                  + jnp.dot(kf, wp_ref[D:, :], preferred_element_type=jnp.float32))


@jax.jit
def kernel(xyz1, xyz2, points1, points2, Wq2, Wk2, Wv, Wproj):
    B, N, _ = xyz1.shape
    M = xyz2.shape[1]
    D = Wq2.shape[0]
    TN = 1024
    grid = (B, N // TN)
    return pl.pallas_call(
        _bi_block_kernel,
        grid=grid,
        in_specs=[
            pl.BlockSpec((1, TN, 3), lambda b, n: (b, n, 0)),
            pl.BlockSpec((1, M, 3), lambda b, n: (b, 0, 0)),
            pl.BlockSpec((1, TN, D), lambda b, n: (b, n, 0)),
            pl.BlockSpec((1, M, D), lambda b, n: (b, 0, 0)),
            pl.BlockSpec((D, D), lambda b, n: (0, 0)),
            pl.BlockSpec((D, D), lambda b, n: (0, 0)),
            pl.BlockSpec((D, D), lambda b, n: (0, 0)),
            pl.BlockSpec((2 * D, D), lambda b, n: (0, 0)),
        ],
        out_specs=pl.BlockSpec((1, TN, D), lambda b, n: (b, n, 0)),
        out_shape=jax.ShapeDtypeStruct((B, N, D), jnp.float32),
        scratch_shapes=[pltpu.VMEM((M, D), jnp.float32),
                        pltpu.VMEM((M, D), jnp.float32)],
    )(xyz1, xyz2, points1, points2, Wq2, Wk2, Wv, Wproj)
